# Initial kernel scaffold; baseline (speedup 1.0000x reference)
import jax
import jax.numpy as jnp
from jax.experimental import pallas as pl


def kernel(event_ts, event_loc, event_flow, pol_mask):
    raise NotImplementedError("write your pallas kernel here")


# speedup vs baseline: 74.2126x; 74.2126x over previous
import functools

import jax
import jax.numpy as jnp
from jax import lax
from jax.experimental import pallas as pl
from jax.experimental.pallas import tpu as pltpu
from jax.experimental.pallas import tpu_sc as plsc

H = 480
W = 640
HW = H * W
ACC = 2 * HW
NTILES = 16
NCORES = 2
BATCHES_PER_CORE = 4
CHUNK = 1792
ENTRIES = 4 * CHUNK
ZCHUNK = 6400
ACC_SLICE = ACC // NTILES


def _splat(ev, batches):
    n_pad = ev.shape[2]
    ev_per_tile = n_pad // NTILES
    nchunk = ev_per_tile // CHUNK
    mesh = plsc.VectorSubcoreMesh(core_axis_name="c", subcore_axis_name="s")

    @functools.partial(
        pl.kernel,
        out_type=jax.ShapeDtypeStruct((batches * ACC,), jnp.float32),
        mesh=mesh,
        scratch_types=[
            pltpu.VMEM((6, CHUNK), jnp.float32),
            pltpu.VMEM((ENTRIES,), jnp.int32),
            pltpu.VMEM((ENTRIES,), jnp.float32),
            pltpu.VMEM((ZCHUNK,), jnp.float32),
            pltpu.VMEM_SHARED((ACC,), jnp.float32),
        ],
    )
    def k(ev_hbm, out_hbm, ev_v, idx_v, val_v, zero_v, acc_sh):
        c = lax.axis_index("c")
        s = lax.axis_index("s")

        @pl.loop(0, ZCHUNK // 16)
        def _(i):
            zero_v[pl.ds(i * 16, 16)] = jnp.zeros((16,), jnp.float32)

        @pl.loop(0, BATCHES_PER_CORE)
        def _(bi):
            b = c * BATCHES_PER_CORE + bi

            @pl.loop(0, ACC_SLICE // ZCHUNK)
            def _(zi):
                pltpu.sync_copy(
                    zero_v, acc_sh.at[pl.ds(s * ACC_SLICE + zi * ZCHUNK, ZCHUNK)])
            plsc.subcore_barrier()

            @pl.loop(0, nchunk)
            def _(ci):
                off = s * ev_per_tile + ci * CHUNK
                pltpu.sync_copy(ev_hbm.at[b, :, pl.ds(off, CHUNK)], ev_v)

                @pl.loop(0, CHUNK // 16)
                def _(vi):
                    sl = pl.ds(vi * 16, 16)
                    ts = ev_v[0, sl]
                    ly = ev_v[1, sl]
                    lx = ev_v[2, sl]
                    fy = ev_v[3, sl]
                    fx = ev_v[4, sl]
                    po = ev_v[5, sl]
                    t = 1.0 - ts
                    wy = ly + t * fy
                    wx = lx + t * fx
                    yi = wy.astype(jnp.int32)
                    ty = jnp.where(yi.astype(jnp.float32) > wy, yi - 1, yi)
                    dy = wy - ty.astype(jnp.float32)
                    xi = wx.astype(jnp.int32)
                    tx = jnp.where(xi.astype(jnp.float32) > wx, xi - 1, xi)
                    dx = wx - tx.astype(jnp.float32)
                    vy0 = (ty >= 0) & (ty <= H - 1)
                    vy1 = (ty >= -1) & (ty <= H - 2)
                    vx0 = (tx >= 0) & (tx <= W - 1)
                    vx1 = (tx >= -1) & (tx <= W - 2)
                    y0 = jnp.minimum(jnp.maximum(ty, 0), H - 1)
                    y1 = jnp.minimum(jnp.maximum(ty + 1, 0), H - 1)
                    x0 = jnp.minimum(jnp.maximum(tx, 0), W - 1)
                    x1 = jnp.minimum(jnp.maximum(tx + 1, 0), W - 1)
                    pz = jnp.minimum(jnp.maximum(po.astype(jnp.int32), 0), 1)
                    row0 = (1 - pz) * HW + y0 * W
                    row1 = (1 - pz) * HW + y1 * W
                    uy = 1.0 - dy
                    ux = 1.0 - dx
                    o = vi * 64
                    idx_v[pl.ds(o, 16)] = row0 + x0
                    val_v[pl.ds(o, 16)] = jnp.where(vy0 & vx0, uy * ux, 0.0)
                    idx_v[pl.ds(o + 16, 16)] = row0 + x1
                    val_v[pl.ds(o + 16, 16)] = jnp.where(vy0 & vx1, uy * dx, 0.0)
                    idx_v[pl.ds(o + 32, 16)] = row1 + x0
                    val_v[pl.ds(o + 32, 16)] = jnp.where(vy1 & vx0, dy * ux, 0.0)
                    idx_v[pl.ds(o + 48, 16)] = row1 + x1
                    val_v[pl.ds(o + 48, 16)] = jnp.where(vy1 & vx1, dy * dx, 0.0)

                pltpu.sync_copy(val_v, acc_sh.at[idx_v], add=True)

            plsc.subcore_barrier()
            pltpu.sync_copy(acc_sh.at[pl.ds(s * ACC_SLICE, ACC_SLICE)],
                            out_hbm.at[pl.ds(b * ACC + s * ACC_SLICE, ACC_SLICE)])

    return k(ev)


def kernel(event_ts, event_loc, event_flow, pol_mask):
    batches, n, _ = event_ts.shape
    ev = jnp.concatenate([
        event_ts[:, :, 0][:, None, :],
        jnp.moveaxis(event_loc, 2, 1),
        jnp.moveaxis(event_flow, 2, 1),
        pol_mask[:, :, 0][:, None, :],
    ], axis=1)
    span = NTILES * CHUNK
    n_pad = ((n + span - 1) // span) * span
    if n_pad != n:
        col = jnp.array([1.0, -10.0, -10.0, 0.0, 0.0, 0.0], jnp.float32)
        pad = jnp.broadcast_to(col[None, :, None], (batches, 6, n_pad - n))
        ev = jnp.concatenate([ev, pad], axis=2)
    out = _splat(ev, batches)
    return out.reshape(batches, 2, H, W)
